# baseline (speedup 1.0000x reference)
import functools

import jax
import jax.numpy as jnp
from jax import lax
from jax.experimental import pallas as pl
from jax.experimental.pallas import tpu as pltpu
from jax.experimental.pallas import tpu_sc as plsc

N = 10000
E = 640000
H = 128

NC = 2
NS = 16
NW = NC * NS
EPW = E // NW
K = 80
NCHUNK = EPW // K
ZROWS = 624


@functools.cache
def _sc_kernels(ne=E):
    mesh = plsc.VectorSubcoreMesh(core_axis_name="c", subcore_axis_name="s")
    EPW = ne // NW
    NCHUNK = EPW // K
    NPAIR = NCHUNK // 2 * 2

    @functools.partial(
        pl.kernel,
        mesh=mesh,
        out_type=[jax.ShapeDtypeStruct((ne, H), jnp.float32),
                  jax.ShapeDtypeStruct((ne, H), jnp.float32)],
        scratch_types=[
            pltpu.VMEM((NCHUNK, K), jnp.int32),
            pltpu.VMEM((NCHUNK, K), jnp.int32),
            pltpu.VMEM((K, H), jnp.float32),
            pltpu.VMEM((K, H), jnp.float32),
            pltpu.VMEM((K, H), jnp.float32),
            pltpu.VMEM((K, H), jnp.float32),
            pltpu.SemaphoreType.DMA,
            pltpu.SemaphoreType.DMA,
            pltpu.SemaphoreType.DMA,
            pltpu.SemaphoreType.DMA,
        ],
    )
    def sc_gather(a_hbm, b_hbm, di_hbm, si_hbm, outa_hbm, outb_hbm,
                  di, si, ar0, br0, ar1, br1,
                  gsem0, gsem1, wsem0, wsem1):
        cid = lax.axis_index("c")
        sid = lax.axis_index("s")
        wid = sid * NC + cid

        pltpu.sync_copy(di_hbm.at[wid], di)
        pltpu.sync_copy(si_hbm.at[wid], si)

        @pl.loop(0, NPAIR, step=2)
        def _(g):
            o0 = wid * EPW + g * K
            o1 = o0 + K
            ca0 = pltpu.async_copy(a_hbm.at[di.at[g]], ar0, gsem0)
            cb0 = pltpu.async_copy(b_hbm.at[si.at[g]], br0, gsem0)
            ca1 = pltpu.async_copy(a_hbm.at[di.at[g + 1]], ar1, gsem1)
            cb1 = pltpu.async_copy(b_hbm.at[si.at[g + 1]], br1, gsem1)
            ca0.wait()
            wa0 = pltpu.async_copy(ar0, outa_hbm.at[pl.ds(o0, K)], wsem0)
            cb0.wait()
            wb0 = pltpu.async_copy(br0, outb_hbm.at[pl.ds(o0, K)], wsem0)
            ca1.wait()
            wa1 = pltpu.async_copy(ar1, outa_hbm.at[pl.ds(o1, K)], wsem1)
            cb1.wait()
            wb1 = pltpu.async_copy(br1, outb_hbm.at[pl.ds(o1, K)], wsem1)
            wa0.wait()
            wb0.wait()
            wa1.wait()
            wb1.wait()

        if NPAIR < NCHUNK:
            g = NCHUNK - 1
            o0 = wid * EPW + g * K
            ca = pltpu.async_copy(a_hbm.at[di.at[g]], ar0, gsem0)
            cb = pltpu.async_copy(b_hbm.at[si.at[g]], br0, gsem0)
            ca.wait()
            cb.wait()
            pltpu.sync_copy(ar0, outa_hbm.at[pl.ds(o0, K)])
            pltpu.sync_copy(br0, outb_hbm.at[pl.ds(o0, K)])

    NHALF = N // 2
    SROWS = 312

    @functools.partial(
        pl.kernel,
        mesh=mesh,
        out_type=jax.ShapeDtypeStruct((NC, 2, NHALF, H), jnp.float32),
        scratch_types=[
            pltpu.VMEM((NCHUNK, K), jnp.int32),
            pltpu.VMEM((K,), jnp.int32),
            pltpu.VMEM((K,), jnp.int32),
            pltpu.VMEM((K, H), jnp.float32),
            pltpu.VMEM((K, H), jnp.float32),
            pltpu.VMEM((8, H), jnp.float32),
            pltpu.VMEM_SHARED((NHALF + 8, H), jnp.float32),
            pltpu.SemaphoreType.DMA,
            pltpu.SemaphoreType.DMA,
            pltpu.SemaphoreType.DMA,
            pltpu.SemaphoreType.DMA,
        ],
    )
    def sc_scatter(msg_hbm, di_hbm, out_hbm,
                   di, idxc0, idxc1, rows0, rows1, zbuf, acc,
                   lsem0, lsem1, asem0, asem1):
        cid = lax.axis_index("c")
        sid = lax.axis_index("s")
        wid = sid * NC + cid

        pltpu.sync_copy(di_hbm.at[wid], di)

        for r in range(8):
            for c in range(H // 16):
                zbuf[r, pl.ds(c * 16, 16)] = jnp.zeros((16,), jnp.float32)

        base = sid * SROWS

        for p in range(2):
            @pl.loop(0, SROWS // 8)
            def _(j):
                pltpu.sync_copy(zbuf, acc.at[pl.ds(base + j * 8, 8)])

            @pl.when(sid == NS - 1)
            def _():
                pltpu.sync_copy(zbuf, acc.at[pl.ds(NS * SROWS, 8)])
                pltpu.sync_copy(zbuf, acc.at[pl.ds(NHALF, 8)])
            plsc.subcore_barrier()

            def clamp(g, idxc):
                for c in range(K // 16):
                    sl = pl.ds(c * 16, 16)
                    v = di[g, sl] - p * NHALF
                    v = jnp.where(v < 0, NHALF, jnp.minimum(v, NHALF))
                    idxc[sl] = v

            @pl.loop(0, NPAIR, step=2)
            def _(g):
                off = wid * EPW + g * K
                l0 = pltpu.async_copy(msg_hbm.at[pl.ds(off, K)], rows0, lsem0)
                l1 = pltpu.async_copy(
                    msg_hbm.at[pl.ds(off + K, K)], rows1, lsem1)
                clamp(g, idxc0)
                l0.wait()
                a0 = pltpu.async_copy(rows0, acc.at[idxc0], asem0, add=True)
                clamp(g + 1, idxc1)
                l1.wait()
                a1 = pltpu.async_copy(rows1, acc.at[idxc1], asem1, add=True)
                a0.wait()
                a1.wait()

            if NPAIR < NCHUNK:
                gt = NCHUNK - 1
                pltpu.sync_copy(msg_hbm.at[pl.ds(wid * EPW + gt * K, K)],
                                rows0)
                clamp(gt, idxc0)
                pltpu.sync_copy(rows0, acc.at[idxc0], add=True)

            plsc.subcore_barrier()

            @pl.loop(0, SROWS // 8)
            def _(j):
                o = base + j * 8
                pltpu.sync_copy(acc.at[pl.ds(o, 8)],
                                out_hbm.at[cid, p, pl.ds(o, 8)])

            @pl.when(sid == NS - 1)
            def _():
                o = NS * SROWS
                pltpu.sync_copy(acc.at[pl.ds(o, 8)],
                                out_hbm.at[cid, p, pl.ds(o, 8)])
            plsc.subcore_barrier()

    return sc_gather, sc_scatter


def _enc_body(x_ref, w1_ref, b1_ref, w2_ref, b2_ref, wt_ref, wb_ref, bl_ref,
              h_ref, a_ref, b_ref):
    h1 = jnp.maximum(
        jnp.dot(x_ref[...], w1_ref[...], preferred_element_type=jnp.float32)
        + b1_ref[...], 0.0)
    h = jnp.maximum(
        jnp.dot(h1, w2_ref[...], preferred_element_type=jnp.float32)
        + b2_ref[...], 0.0)
    h_ref[...] = h
    a_ref[...] = jnp.dot(h, wt_ref[...],
                         preferred_element_type=jnp.float32) + bl_ref[...]
    b_ref[...] = jnp.dot(h, wb_ref[...], preferred_element_type=jnp.float32)


def _ab_body(h_ref, p_ref, q_ref, wt_ref, wb_ref, bl_ref,
             hn_ref, a_ref, b_ref):
    ps = p_ref[...] + q_ref[...]
    agg = ps[0] + ps[1]
    hn = h_ref[...] + jnp.concatenate([agg[0], agg[1]], axis=0)
    hn_ref[...] = hn
    a_ref[...] = jnp.dot(hn, wt_ref[...],
                         preferred_element_type=jnp.float32) + bl_ref[...]
    b_ref[...] = jnp.dot(hn, wb_ref[...], preferred_element_type=jnp.float32)


def _mlp_body(a_ref, b_ref, w_ref, bias_ref, o_ref):
    m = jnp.maximum(a_ref[...] + b_ref[...], 0.0)
    o_ref[...] = jnp.maximum(
        jnp.dot(m, w_ref[...], preferred_element_type=jnp.float32)
        + bias_ref[...], 0.0)


def _dec_body(h_ref, p_ref, q_ref, w1_ref, b1_ref, w2_ref, b2_ref, o_ref):
    ps = p_ref[...] + q_ref[...]
    agg = ps[0] + ps[1]
    hn = h_ref[...] + jnp.concatenate([agg[0], agg[1]], axis=0)
    d1 = jnp.maximum(
        jnp.dot(hn, w1_ref[...], preferred_element_type=jnp.float32)
        + b1_ref[...], 0.0)
    o_ref[...] = jnp.dot(d1, w2_ref[...],
                         preferred_element_type=jnp.float32) + b2_ref[...]


_BE = 5000


def _mlp_call(m1a, m1b, w2, b2):
    ne = m1a.shape[0]
    return pl.pallas_call(
        _mlp_body,
        grid=(ne // _BE,),
        in_specs=[
            pl.BlockSpec((_BE, H), lambda i: (i, 0)),
            pl.BlockSpec((_BE, H), lambda i: (i, 0)),
            pl.BlockSpec((H, H), lambda i: (0, 0)),
            pl.BlockSpec((1, H), lambda i: (0, 0)),
        ],
        out_specs=pl.BlockSpec((_BE, H), lambda i: (i, 0)),
        out_shape=jax.ShapeDtypeStruct((ne, H), jnp.float32),
    )(m1a, m1b, w2, b2)


def kernel(x, edge_index, enc_w1, enc_b1, enc_w2, enc_b2,
           l0_w1, l0_b1, l0_w2, l0_b2,
           l1_w1, l1_b1, l1_w2, l1_b2,
           l2_w1, l2_b1, l2_w2, l2_b2,
           dec_w1, dec_b1, dec_w2, dec_b2):
    sc_gather, sc_scatter = _sc_kernels(E // 2)

    src = edge_index[0]
    dst = edge_index[1]
    EH = E // 2
    NCH = EH // NW // K
    dsti1 = dst[:EH].reshape(NW, NCH, K)
    srci1 = src[:EH].reshape(NW, NCH, K)
    dsti2 = dst[EH:].reshape(NW, NCH, K)
    srci2 = src[EH:].reshape(NW, NCH, K)

    x8 = jnp.pad(x, ((0, 0), (0, 1)))
    ew1 = jnp.pad(enc_w1, ((0, 1), (0, 0)))
    eb1 = enc_b1.reshape(1, H)
    eb2 = enc_b2.reshape(1, H)
    dw2 = jnp.pad(dec_w2, ((0, 0), (0, 2)))
    db1 = dec_b1.reshape(1, H)
    db2 = jnp.pad(dec_b2, (0, 2)).reshape(1, 8)

    layers = [(l0_w1, l0_b1, l0_w2, l0_b2),
              (l1_w1, l1_b1, l1_w2, l1_b2),
              (l2_w1, l2_b1, l2_w2, l2_b2)]
    splits = [(w1[:H], w1[H:], b1.reshape(1, H)) for (w1, b1, _, _) in layers]

    nh = jax.ShapeDtypeStruct((N, H), jnp.float32)
    h, A, B = pl.pallas_call(
        _enc_body,
        out_shape=[nh, nh, nh],
    )(x8, ew1, eb1, enc_w2, eb2, splits[0][0], splits[0][1], splits[0][2])

    Pa = Pb = None
    for li in range(3):
        _, _, w2, b2 = layers[li]
        b2r = b2.reshape(1, H)
        ra1, rb1 = sc_gather(A, B, dsti1, srci1)
        ra2, rb2 = sc_gather(A, B, dsti2, srci2)
        msga = _mlp_call(ra1, rb1, w2, b2r)
        msgb = _mlp_call(ra2, rb2, w2, b2r)
        Pa = sc_scatter(msga, dsti1)
        Pb = sc_scatter(msgb, dsti2)
        if li < 2:
            wt, wb, bl = splits[li + 1]
            h, A, B = pl.pallas_call(
                _ab_body,
                out_shape=[nh, nh, nh],
            )(h, Pa, Pb, wt, wb, bl)

    out = pl.pallas_call(
        _dec_body,
        out_shape=jax.ShapeDtypeStruct((N, 8), jnp.float32),
    )(h, Pa, Pb, dec_w1, db1, dw2, db2)
    return (out[:, :3], out[:, 3:6])


# speedup vs baseline: 1.0339x; 1.0339x over previous
import functools

import jax
import jax.numpy as jnp
from jax import lax
from jax.experimental import pallas as pl
from jax.experimental.pallas import tpu as pltpu
from jax.experimental.pallas import tpu_sc as plsc

N = 10000
E = 640000
H = 128

NC = 2
NS = 16
NW = NC * NS
EPW = E // NW
K = 80
NCHUNK = EPW // K
ZROWS = 624


@functools.cache
def _sc_kernels(ne=E):
    mesh = plsc.VectorSubcoreMesh(core_axis_name="c", subcore_axis_name="s")
    EPW = ne // NW
    NCHUNK = EPW // K
    NPAIR = NCHUNK // 2 * 2

    @functools.partial(
        pl.kernel,
        mesh=mesh,
        out_type=jax.ShapeDtypeStruct((ne, H), jnp.float32),
        scratch_types=[
            pltpu.VMEM((NCHUNK, K), jnp.int32),
            pltpu.VMEM((NCHUNK, K), jnp.int32),
            pltpu.VMEM((K, H), jnp.float32),
            pltpu.VMEM((K, H), jnp.float32),
            pltpu.VMEM((K, H), jnp.float32),
            pltpu.VMEM((K, H), jnp.float32),
            pltpu.VMEM((K, H), jnp.float32),
            pltpu.VMEM((K, H), jnp.float32),
            pltpu.SemaphoreType.DMA,
            pltpu.SemaphoreType.DMA,
            pltpu.SemaphoreType.DMA,
            pltpu.SemaphoreType.DMA,
        ],
    )
    def sc_gather(a_hbm, b_hbm, di_hbm, si_hbm, out_hbm,
                  di, si, ar0, br0, mr0, ar1, br1, mr1,
                  gsem0, gsem1, wsem0, wsem1):
        cid = lax.axis_index("c")
        sid = lax.axis_index("s")
        wid = sid * NC + cid

        pltpu.sync_copy(di_hbm.at[wid], di)
        pltpu.sync_copy(si_hbm.at[wid], si)

        def compute(ar, br, mr):
            def row(r, carry):
                for c in range(H // 16):
                    sl = pl.ds(c * 16, 16)
                    mr[r, sl] = jnp.maximum(ar[r, sl] + br[r, sl], 0.0)
                return carry
            lax.fori_loop(0, K, row, 0)

        @pl.loop(0, NPAIR, step=2)
        def _(g):
            ca0 = pltpu.async_copy(a_hbm.at[di.at[g]], ar0, gsem0)
            cb0 = pltpu.async_copy(b_hbm.at[si.at[g]], br0, gsem0)
            ca1 = pltpu.async_copy(a_hbm.at[di.at[g + 1]], ar1, gsem1)
            cb1 = pltpu.async_copy(b_hbm.at[si.at[g + 1]], br1, gsem1)
            ca0.wait()
            cb0.wait()
            compute(ar0, br0, mr0)
            w0 = pltpu.async_copy(
                mr0, out_hbm.at[pl.ds(wid * EPW + g * K, K)], wsem0)
            ca1.wait()
            cb1.wait()
            compute(ar1, br1, mr1)
            w1 = pltpu.async_copy(
                mr1, out_hbm.at[pl.ds(wid * EPW + (g + 1) * K, K)], wsem1)
            w0.wait()
            w1.wait()

        if NPAIR < NCHUNK:
            g = NCHUNK - 1
            ca = pltpu.async_copy(a_hbm.at[di.at[g]], ar0, gsem0)
            cb = pltpu.async_copy(b_hbm.at[si.at[g]], br0, gsem0)
            ca.wait()
            cb.wait()
            compute(ar0, br0, mr0)
            pltpu.sync_copy(mr0, out_hbm.at[pl.ds(wid * EPW + g * K, K)])

    NHALF = N // 2
    SROWS = 312

    @functools.partial(
        pl.kernel,
        mesh=mesh,
        out_type=jax.ShapeDtypeStruct((NC, 2, NHALF, H), jnp.float32),
        scratch_types=[
            pltpu.VMEM((NCHUNK, K), jnp.int32),
            pltpu.VMEM((K,), jnp.int32),
            pltpu.VMEM((K,), jnp.int32),
            pltpu.VMEM((K, H), jnp.float32),
            pltpu.VMEM((K, H), jnp.float32),
            pltpu.VMEM((8, H), jnp.float32),
            pltpu.VMEM_SHARED((NHALF + 8, H), jnp.float32),
            pltpu.SemaphoreType.DMA,
            pltpu.SemaphoreType.DMA,
            pltpu.SemaphoreType.DMA,
            pltpu.SemaphoreType.DMA,
        ],
    )
    def sc_scatter(msg_hbm, di_hbm, out_hbm,
                   di, idxc0, idxc1, rows0, rows1, zbuf, acc,
                   lsem0, lsem1, asem0, asem1):
        cid = lax.axis_index("c")
        sid = lax.axis_index("s")
        wid = sid * NC + cid

        pltpu.sync_copy(di_hbm.at[wid], di)

        for r in range(8):
            for c in range(H // 16):
                zbuf[r, pl.ds(c * 16, 16)] = jnp.zeros((16,), jnp.float32)

        base = sid * SROWS

        for p in range(2):
            @pl.loop(0, SROWS // 8)
            def _(j):
                pltpu.sync_copy(zbuf, acc.at[pl.ds(base + j * 8, 8)])

            @pl.when(sid == NS - 1)
            def _():
                pltpu.sync_copy(zbuf, acc.at[pl.ds(NS * SROWS, 8)])
                pltpu.sync_copy(zbuf, acc.at[pl.ds(NHALF, 8)])
            plsc.subcore_barrier()

            def clamp(g, idxc):
                for c in range(K // 16):
                    sl = pl.ds(c * 16, 16)
                    v = di[g, sl] - p * NHALF
                    v = jnp.where(v < 0, NHALF, jnp.minimum(v, NHALF))
                    idxc[sl] = v

            @pl.loop(0, NPAIR, step=2)
            def _(g):
                off = wid * EPW + g * K
                l0 = pltpu.async_copy(msg_hbm.at[pl.ds(off, K)], rows0, lsem0)
                l1 = pltpu.async_copy(
                    msg_hbm.at[pl.ds(off + K, K)], rows1, lsem1)
                clamp(g, idxc0)
                l0.wait()
                a0 = pltpu.async_copy(rows0, acc.at[idxc0], asem0, add=True)
                clamp(g + 1, idxc1)
                l1.wait()
                a1 = pltpu.async_copy(rows1, acc.at[idxc1], asem1, add=True)
                a0.wait()
                a1.wait()

            if NPAIR < NCHUNK:
                gt = NCHUNK - 1
                pltpu.sync_copy(msg_hbm.at[pl.ds(wid * EPW + gt * K, K)],
                                rows0)
                clamp(gt, idxc0)
                pltpu.sync_copy(rows0, acc.at[idxc0], add=True)

            plsc.subcore_barrier()

            @pl.loop(0, SROWS // 8)
            def _(j):
                o = base + j * 8
                pltpu.sync_copy(acc.at[pl.ds(o, 8)],
                                out_hbm.at[cid, p, pl.ds(o, 8)])

            @pl.when(sid == NS - 1)
            def _():
                o = NS * SROWS
                pltpu.sync_copy(acc.at[pl.ds(o, 8)],
                                out_hbm.at[cid, p, pl.ds(o, 8)])
            plsc.subcore_barrier()

    return sc_gather, sc_scatter


def _enc_body(x_ref, w1_ref, b1_ref, w2_ref, b2_ref, wt_ref, wb_ref, bl_ref,
              h_ref, a_ref, b_ref):
    h1 = jnp.maximum(
        jnp.dot(x_ref[...], w1_ref[...], preferred_element_type=jnp.float32)
        + b1_ref[...], 0.0)
    h = jnp.maximum(
        jnp.dot(h1, w2_ref[...], preferred_element_type=jnp.float32)
        + b2_ref[...], 0.0)
    h_ref[...] = h
    a_ref[...] = jnp.dot(h, wt_ref[...],
                         preferred_element_type=jnp.float32) + bl_ref[...]
    b_ref[...] = jnp.dot(h, wb_ref[...], preferred_element_type=jnp.float32)


def _ab_body(h_ref, p_ref, q_ref, wt_ref, wb_ref, bl_ref,
             hn_ref, a_ref, b_ref):
    ps = p_ref[...] + q_ref[...]
    agg = ps[0] + ps[1]
    hn = h_ref[...] + jnp.concatenate([agg[0], agg[1]], axis=0)
    hn_ref[...] = hn
    a_ref[...] = jnp.dot(hn, wt_ref[...],
                         preferred_element_type=jnp.float32) + bl_ref[...]
    b_ref[...] = jnp.dot(hn, wb_ref[...], preferred_element_type=jnp.float32)


def _mlp_body(m_ref, w_ref, b_ref, o_ref):
    o_ref[...] = jnp.maximum(
        jnp.dot(m_ref[...], w_ref[...], preferred_element_type=jnp.float32)
        + b_ref[...], 0.0)


def _dec_body(h_ref, p_ref, q_ref, w1_ref, b1_ref, w2_ref, b2_ref, o_ref):
    ps = p_ref[...] + q_ref[...]
    agg = ps[0] + ps[1]
    hn = h_ref[...] + jnp.concatenate([agg[0], agg[1]], axis=0)
    d1 = jnp.maximum(
        jnp.dot(hn, w1_ref[...], preferred_element_type=jnp.float32)
        + b1_ref[...], 0.0)
    o_ref[...] = jnp.dot(d1, w2_ref[...],
                         preferred_element_type=jnp.float32) + b2_ref[...]


_BE = 5000


def _mlp_call(m1, w2, b2):
    ne = m1.shape[0]
    return pl.pallas_call(
        _mlp_body,
        grid=(ne // _BE,),
        in_specs=[
            pl.BlockSpec((_BE, H), lambda i: (i, 0)),
            pl.BlockSpec((H, H), lambda i: (0, 0)),
            pl.BlockSpec((1, H), lambda i: (0, 0)),
        ],
        out_specs=pl.BlockSpec((_BE, H), lambda i: (i, 0)),
        out_shape=jax.ShapeDtypeStruct((ne, H), jnp.float32),
    )(m1, w2, b2)


def kernel(x, edge_index, enc_w1, enc_b1, enc_w2, enc_b2,
           l0_w1, l0_b1, l0_w2, l0_b2,
           l1_w1, l1_b1, l1_w2, l1_b2,
           l2_w1, l2_b1, l2_w2, l2_b2,
           dec_w1, dec_b1, dec_w2, dec_b2):
    sc_gather, sc_scatter = _sc_kernels(E // 2)

    src = edge_index[0]
    dst = edge_index[1]
    EH = E // 2
    NCH = EH // NW // K
    dsti1 = dst[:EH].reshape(NW, NCH, K)
    srci1 = src[:EH].reshape(NW, NCH, K)
    dsti2 = dst[EH:].reshape(NW, NCH, K)
    srci2 = src[EH:].reshape(NW, NCH, K)

    x8 = jnp.pad(x, ((0, 0), (0, 1)))
    ew1 = jnp.pad(enc_w1, ((0, 1), (0, 0)))
    eb1 = enc_b1.reshape(1, H)
    eb2 = enc_b2.reshape(1, H)
    dw2 = jnp.pad(dec_w2, ((0, 0), (0, 2)))
    db1 = dec_b1.reshape(1, H)
    db2 = jnp.pad(dec_b2, (0, 2)).reshape(1, 8)

    layers = [(l0_w1, l0_b1, l0_w2, l0_b2),
              (l1_w1, l1_b1, l1_w2, l1_b2),
              (l2_w1, l2_b1, l2_w2, l2_b2)]
    splits = [(w1[:H], w1[H:], b1.reshape(1, H)) for (w1, b1, _, _) in layers]

    nh = jax.ShapeDtypeStruct((N, H), jnp.float32)
    h, A, B = pl.pallas_call(
        _enc_body,
        out_shape=[nh, nh, nh],
    )(x8, ew1, eb1, enc_w2, eb2, splits[0][0], splits[0][1], splits[0][2])

    Pa = Pb = None
    for li in range(3):
        _, _, w2, b2 = layers[li]
        b2r = b2.reshape(1, H)
        m1a = sc_gather(A, B, dsti1, srci1)
        m1b = sc_gather(A, B, dsti2, srci2)
        msga = _mlp_call(m1a, w2, b2r)
        msgb = _mlp_call(m1b, w2, b2r)
        Pa = sc_scatter(msga, dsti1)
        Pb = sc_scatter(msgb, dsti2)
        if li < 2:
            wt, wb, bl = splits[li + 1]
            h, A, B = pl.pallas_call(
                _ab_body,
                out_shape=[nh, nh, nh],
            )(h, Pa, Pb, wt, wb, bl)

    out = pl.pallas_call(
        _dec_body,
        out_shape=jax.ShapeDtypeStruct((N, 8), jnp.float32),
    )(h, Pa, Pb, dec_w1, db1, dw2, db2)
    return (out[:, :3], out[:, 3:6])
